# Initial kernel scaffold; baseline (speedup 1.0000x reference)
import jax
import jax.numpy as jnp
from jax.experimental import pallas as pl


def kernel(v0, v1, v2, c0, c1, c2):
    raise NotImplementedError("write your pallas kernel here")


# speedup vs baseline: 1.1009x; 1.1009x over previous
import functools

import jax
import jax.numpy as jnp
from jax.experimental import pallas as pl
from jax.experimental.pallas import tpu as pltpu

def _vq_tile_kernel(v_ref, c_ref, minval_ref, argmin_ref, losssum_ref, *, bk, nk):
    j = pl.program_id(0)
    v = v_ref[...]
    c = c_ref[...]
    cross = jax.lax.dot_general(
        v, c, (((1,), (1,)), ((), ())),
        preferred_element_type=jnp.float32)
    v2 = jnp.sum(v * v, axis=1)
    c2 = jnp.sum(c * c, axis=1)
    dist = (v2[:, None] + c2[None, :]) - 2.0 * cross

    local_min = jnp.min(dist, axis=1)
    lane = jax.lax.broadcasted_iota(jnp.int32, dist.shape, 1) + j * bk
    local_arg = jnp.min(
        jnp.where(dist == local_min[:, None], lane, 2147483647), axis=1)

    @pl.when(j == 0)
    def _init():
        minval_ref[...] = local_min
        argmin_ref[...] = local_arg

    @pl.when(j > 0)
    def _update():
        prev = minval_ref[...]
        better = local_min < prev
        minval_ref[...] = jnp.where(better, local_min, prev)
        argmin_ref[...] = jnp.where(better, local_arg, argmin_ref[...])

    @pl.when(j == nk - 1)
    def _finalize():
        losssum_ref[0, 0] = jnp.sum(minval_ref[...])


def _vq_assign(v2d, c, bk):
    m, d = v2d.shape
    k = c.shape[0]
    nk = k // bk
    minval, argmin, losssum = pl.pallas_call(
        functools.partial(_vq_tile_kernel, bk=bk, nk=nk),
        grid=(nk,),
        in_specs=[
            pl.BlockSpec((m, d), lambda j: (0, 0)),
            pl.BlockSpec((bk, d), lambda j: (j, 0)),
        ],
        out_specs=[
            pl.BlockSpec((m,), lambda j: (0,)),
            pl.BlockSpec((m,), lambda j: (0,)),
            pl.BlockSpec(memory_space=pltpu.SMEM),
        ],
        out_shape=[
            jax.ShapeDtypeStruct((m,), jnp.float32),
            jax.ShapeDtypeStruct((m,), jnp.int32),
            jax.ShapeDtypeStruct((1, 1), jnp.float32),
        ],
        compiler_params=pltpu.CompilerParams(
            dimension_semantics=("arbitrary",)),
    )(v2d, c)
    return minval, argmin, losssum[0, 0]


def kernel(v0, v1, v2, c0, c1, c2):
    b, s, d = v0.shape
    m = b * s
    outs = []
    for v, c in ((v0, c0), (v1, c1), (v2, c2)):
        outs.append(_vq_assign(v.reshape(m, d), c, bk=512))
    losses = jnp.stack([o[2] for o in outs]) / jnp.float32(m)
    loss = jnp.mean(losses)
    a0, a1, a2 = (o[1].reshape(b, s) for o in outs)
    return (loss, a0, a1, a2)


# speedup vs baseline: 1.6785x; 1.5247x over previous
import functools

import jax
import jax.numpy as jnp
from jax.experimental import pallas as pl
from jax.experimental.pallas import tpu as pltpu


def _vq_tile_kernel(v_ref, c_ref, argmin_ref, losssum_ref,
                    runval_ref, runchunk_ref, *, bk, nk):
    j = pl.program_id(0)
    m = v_ref.shape[0]
    v = v_ref[...]
    c = c_ref[...]
    cross2 = jax.lax.dot_general(
        v, c + c, (((1,), (1,)), ((), ())),
        preferred_element_type=jnp.float32)
    v2 = jnp.sum(v * v, axis=1)
    c2 = jnp.sum(c * c, axis=1)

    @pl.when(j == 0)
    def _init():
        runval_ref[...] = jnp.full((m, 128), jnp.inf, jnp.float32)
        runchunk_ref[...] = jnp.zeros((m, 128), jnp.int32)

    for t in range(bk // 128):
        sl = slice(t * 128, (t + 1) * 128)
        dist = (v2[:, None] + c2[None, sl]) - cross2[:, sl]
        better = dist < runval_ref[...]
        runval_ref[...] = jnp.minimum(runval_ref[...], dist)
        chunkno = j * (bk // 128) + t
        runchunk_ref[...] = jnp.where(
            better, jnp.int32(chunkno), runchunk_ref[...])

    @pl.when(j == nk - 1)
    def _finalize():
        runval = runval_ref[...]
        gmin = jnp.min(runval, axis=1)
        lane = jax.lax.broadcasted_iota(jnp.int32, (m, 128), 1)
        kidx = runchunk_ref[...] * 128 + lane
        cand = jnp.where(runval == gmin[:, None], kidx, 2147483647)
        argmin_ref[...] = jnp.min(cand, axis=1)
        losssum_ref[0, 0] = jnp.sum(gmin)


def _vq_assign(v2d, c, bk):
    m, d = v2d.shape
    k = c.shape[0]
    nk = k // bk
    argmin, losssum = pl.pallas_call(
        functools.partial(_vq_tile_kernel, bk=bk, nk=nk),
        grid=(nk,),
        in_specs=[
            pl.BlockSpec((m, d), lambda j: (0, 0)),
            pl.BlockSpec((bk, d), lambda j: (j, 0)),
        ],
        out_specs=[
            pl.BlockSpec((m,), lambda j: (0,)),
            pl.BlockSpec(memory_space=pltpu.SMEM),
        ],
        out_shape=[
            jax.ShapeDtypeStruct((m,), jnp.int32),
            jax.ShapeDtypeStruct((1, 1), jnp.float32),
        ],
        scratch_shapes=[
            pltpu.VMEM((m, 128), jnp.float32),
            pltpu.VMEM((m, 128), jnp.int32),
        ],
        compiler_params=pltpu.CompilerParams(
            dimension_semantics=("arbitrary",)),
    )(v2d, c)
    return argmin, losssum[0, 0]


def kernel(v0, v1, v2, c0, c1, c2):
    b, s, d = v0.shape
    m = b * s
    outs = []
    for v, c in ((v0, c0), (v1, c1), (v2, c2)):
        outs.append(_vq_assign(v.reshape(m, d), c, bk=512))
    losses = jnp.stack([o[1] for o in outs]) / jnp.float32(m)
    loss = jnp.mean(losses)
    a0, a1, a2 = (o[0].reshape(b, s) for o in outs)
    return (loss, a0, a1, a2)


# speedup vs baseline: 2.2639x; 1.3487x over previous
import functools

import jax
import jax.numpy as jnp
from jax.experimental import pallas as pl
from jax.experimental.pallas import tpu as pltpu


def _vq_tile_kernel(v_ref, c_ref, argmin_ref, losssum_ref,
                    runval_ref, runchunk_ref, v2_ref, *, bk, nk):
    j = pl.program_id(0)
    m = v_ref.shape[0]
    v = v_ref[...]
    c = c_ref[...]
    cross2 = jax.lax.dot_general(
        v, c + c, (((1,), (1,)), ((), ())),
        preferred_element_type=jnp.float32)
    c2 = jnp.sum(c * c, axis=1)

    @pl.when(j == 0)
    def _init():
        v2 = jnp.sum(v * v, axis=1)
        v2_ref[...] = jnp.broadcast_to(v2[:, None], (m, 128))
        runval_ref[...] = jnp.full((m, 128), jnp.inf, jnp.float32)
        runchunk_ref[...] = jnp.zeros((m, 128), jnp.int32)

    v2b = v2_ref[...]
    rv = runval_ref[...]
    rc = runchunk_ref[...]
    for t in range(bk // 128):
        sl = slice(t * 128, (t + 1) * 128)
        dist = (v2b + c2[None, sl]) - cross2[:, sl]
        better = dist < rv
        rv = jnp.minimum(rv, dist)
        chunkno = j * (bk // 128) + t
        rc = jnp.where(better, jnp.int32(chunkno), rc)
    runval_ref[...] = rv
    runchunk_ref[...] = rc

    @pl.when(j == nk - 1)
    def _finalize():
        gmin = jnp.min(rv, axis=1)
        lane = jax.lax.broadcasted_iota(jnp.int32, (m, 128), 1)
        kidx = rc * 128 + lane
        cand = jnp.where(rv == gmin[:, None], kidx, 2147483647)
        argmin_ref[...] = jnp.min(cand, axis=1)
        losssum_ref[0, 0] = jnp.sum(gmin)


def _vq_assign(v2d, c, bk):
    m, d = v2d.shape
    k = c.shape[0]
    bk = min(bk, k)
    nk = k // bk
    argmin, losssum = pl.pallas_call(
        functools.partial(_vq_tile_kernel, bk=bk, nk=nk),
        grid=(nk,),
        in_specs=[
            pl.BlockSpec((m, d), lambda j: (0, 0)),
            pl.BlockSpec((bk, d), lambda j: (j, 0)),
        ],
        out_specs=[
            pl.BlockSpec((m,), lambda j: (0,)),
            pl.BlockSpec(memory_space=pltpu.SMEM),
        ],
        out_shape=[
            jax.ShapeDtypeStruct((m,), jnp.int32),
            jax.ShapeDtypeStruct((1, 1), jnp.float32),
        ],
        scratch_shapes=[
            pltpu.VMEM((m, 128), jnp.float32),
            pltpu.VMEM((m, 128), jnp.int32),
            pltpu.VMEM((m, 128), jnp.float32),
        ],
        compiler_params=pltpu.CompilerParams(
            dimension_semantics=("arbitrary",)),
    )(v2d, c)
    return argmin, losssum[0, 0]


def kernel(v0, v1, v2, c0, c1, c2):
    b, s, d = v0.shape
    m = b * s
    outs = []
    for v, c in ((v0, c0), (v1, c1), (v2, c2)):
        outs.append(_vq_assign(v.reshape(m, d), c, bk=1024))
    losses = jnp.stack([o[1] for o in outs]) / jnp.float32(m)
    loss = jnp.mean(losses)
    a0, a1, a2 = (o[0].reshape(b, s) for o in outs)
    return (loss, a0, a1, a2)
